# baseline (speedup 1.0000x reference)
import functools

import jax
import jax.numpy as jnp
from jax import lax
from jax.experimental import pallas as pl
from jax.experimental.pallas import tpu as pltpu
from jax.experimental.pallas import tpu_sc as plsc

L = 16
NUM_WORKERS = 32
CHUNK_B = 32
IDX_PER_STREAM = 128
VBLK = 768
VTAIL = 32


def _retile_kernel(H, V, R, tabT_hbm, tail_hbm, out_hbm, ib0, ib1, ob0, ob1,
                   tail_vm, sem_in, sem_out):
    n_full = (V - V % VBLK) // VBLK
    v_mid = n_full * VBLK
    SR = VBLK * R // 128
    n_fb = H * n_full
    n_iter = -(-n_fb // NUM_WORKERS)
    wid = lax.axis_index("s") * 2 + lax.axis_index("c")
    iota = lax.iota(jnp.int32, L)
    ibs = [ib0, ib1]
    obs = [ob0, ob1]

    pltpu.sync_copy(tail_hbm, tail_vm)

    rvecs = [((cp * L + iota) & (R - 1)) for cp in range(8)]
    dvbs = [lax.shift_right_logical(cp * L + iota, 5) for cp in range(8)]

    def issue_in(blk, slot):
        m = blk // n_full
        t = blk % n_full
        mr0 = pl.multiple_of(m * R, 8)
        v0 = pl.multiple_of(t * VBLK, 128)
        return pltpu.async_copy(
            tabT_hbm.at[pl.ds(mr0, R), pl.ds(v0, VBLK)],
            ibs[slot].at[:, pl.ds(0, VBLK)], sem_in)

    def drain_in(slot):
        pltpu.make_async_copy(
            tabT_hbm.at[pl.ds(0, R), pl.ds(0, VBLK)],
            ibs[slot].at[:, pl.ds(0, VBLK)], sem_in).wait()

    def drain_out(slot):
        pltpu.make_async_copy(
            obs[slot], out_hbm.at[pl.ds(0, SR), :], sem_out).wait()

    @pl.when(wid < n_fb)
    def _prologue():
        issue_in(wid, 0)

    def pair(q):
        for par in range(2):
            k = 2 * q + par
            blk = wid + k * NUM_WORKERS

            @pl.when(blk < n_fb)
            def _(par=par, k=k, blk=blk):
                nxt = blk + NUM_WORKERS

                @pl.when(nxt < n_fb)
                def _prefetch():
                    issue_in(nxt, par ^ 1)

                drain_in(par)
                ib = ibs[par]
                ob = obs[par]

                @pl.when(k >= 2)
                def _reclaim():
                    drain_out(par)

                def srow8(q):
                    s8 = q * 8
                    for u in range(8):
                        for cp in range(8):
                            ob[s8 + u, pl.ds(cp * L, L)] = plsc.load_gather(
                                ib, [rvecs[cp], dvbs[cp] + 4 * (s8 + u)])

                pl.loop(0, SR // 8)(srow8)
                m = blk // n_full
                t = blk % n_full
                s0 = pl.multiple_of((m * V + t * VBLK) // 4, 8)
                pltpu.async_copy(ob, out_hbm.at[pl.ds(s0, SR), :], sem_out)

    pl.loop(0, -(-n_iter // 2))(pair)

    n_mine = (n_fb - wid + NUM_WORKERS - 1) // NUM_WORKERS

    @pl.when(n_mine >= 2)
    def _d2():
        drain_out(0)

    @pl.when(n_mine >= 1)
    def _d1():
        drain_out(0)

    @pl.when(wid < H)
    def _rest():
        m = wid
        mr0 = pl.multiple_of(m * R, 8)

        v0 = pl.multiple_of(v_mid, 128)
        pltpu.async_copy(tabT_hbm.at[pl.ds(mr0, R), pl.ds(v0, 128)],
                         ib0.at[:, pl.ds(0, 128)], sem_in).wait()

        def srow_mid(s):
            for cp in range(8):
                ob0[s, pl.ds(cp * L, L)] = plsc.load_gather(
                    ib0, [rvecs[cp], dvbs[cp] + 4 * s])

        pl.loop(0, 32)(srow_mid)
        s0 = pl.multiple_of((m * V + v_mid) // 4, 8)
        pltpu.async_copy(ob0.at[pl.ds(0, 32), :],
                         out_hbm.at[pl.ds(s0, 32), :], sem_out).wait()

        def srow_tail(s):
            for cp in range(8):
                e = (m * R + rvecs[cp]) * VTAIL + dvbs[cp] + 4 * s
                ob1[s, pl.ds(cp * L, L)] = plsc.load_gather(
                    tail_vm, [lax.shift_right_logical(e, 7), e & 127])

        pl.loop(0, VTAIL * R // 128)(srow_tail)
        s1 = pl.multiple_of((m * V + V - VTAIL) // 4, 8)
        pltpu.async_copy(ob1.at[pl.ds(0, VTAIL * R // 128), :],
                         out_hbm.at[pl.ds(s1, VTAIL * R // 128), :],
                         sem_out).wait()


def _cp_kernel(H, V, R, B, coords_hbm, offs_hbm, factors_hbm, weights_hbm,
               bias_hbm, out_hbm, cvm, offs_vm, fidx_vm, subs_vm, rows_vm,
               wvm, bvm, out_vm, sem):
    per_tile = B // NUM_WORKERS
    n_chunks = per_tile // CHUNK_B
    idx_per_chunk = CHUNK_B * H
    n_streams = idx_per_chunk // IDX_PER_STREAM
    n_groups = CHUNK_B // L

    wid = lax.axis_index("s") * 2 + lax.axis_index("c")

    pltpu.sync_copy(offs_hbm, offs_vm)
    pltpu.sync_copy(weights_hbm, wvm)
    pltpu.sync_copy(bias_hbm, bvm)

    iota = lax.iota(jnp.int32, L)
    iotaH = iota * H
    bias_vec = bvm[...]

    def chunk_body(c):
        base = (wid * per_tile + c * CHUNK_B) * H
        pltpu.sync_copy(coords_hbm.at[pl.ds(base, idx_per_chunk)], cvm)
        for i in range(idx_per_chunk // L):
            j, col = (i * L) // IDX_PER_STREAM, (i * L) % IDX_PER_STREAM
            sl = pl.ds(i * L, L)
            flat = cvm[sl] + offs_vm[sl]
            fidx_vm[j, pl.ds(col, L)] = lax.shift_right_logical(flat, 2)
            subs_vm[sl] = lax.shift_left(flat & 3, 5)
        copies = []
        for j in range(n_streams):
            copies.append(
                pltpu.async_copy(
                    factors_hbm.at[fidx_vm.at[j]],
                    rows_vm.at[pl.ds(j * IDX_PER_STREAM, IDX_PER_STREAM)],
                    sem,
                ))
        for cp in copies:
            cp.wait()

        def group_body(g):
            row0 = g * (L * H) + iotaH
            subv = [plsc.load_gather(subs_vm, [row0 + m]) for m in range(H)]
            out_vec = bias_vec
            for r in range(R):
                colrot = (iota + r) & (R - 1)
                acc = plsc.load_gather(rows_vm, [row0, subv[0] + colrot])
                for m in range(1, H):
                    acc = acc * plsc.load_gather(rows_vm,
                                                 [row0 + m, subv[m] + colrot])
                out_vec = out_vec + acc * plsc.load_gather(wvm, [colrot])
            out_vm[pl.ds(c * CHUNK_B + g * L, L)] = out_vec

        pl.loop(0, n_groups)(group_body)

    pl.loop(0, n_chunks)(chunk_body)
    pltpu.sync_copy(out_vm, out_hbm.at[pl.ds(wid * per_tile, per_tile)])


@jax.jit
def kernel(coords, factors, weights, bias):
    B, H = coords.shape
    _, V, R = factors.shape
    assert R == 32 and B % NUM_WORKERS == 0 and V % 4 == 0
    per_tile = B // NUM_WORKERS

    mesh = plsc.VectorSubcoreMesh(core_axis_name="c", subcore_axis_name="s",
                                  num_cores=2, num_subcores=16)
    params = pltpu.CompilerParams(needs_layout_passes=False,
                                  use_tc_tiling_on_sc=True)

    tabT = jnp.transpose(factors, (0, 2, 1)).reshape(H * R, V)
    tail = jnp.transpose(factors[:, V - VTAIL:, :], (0, 2, 1)).reshape(
        H * R * VTAIL // 128, 128)
    retile = pl.kernel(
        functools.partial(_retile_kernel, H, V, R),
        out_type=jax.ShapeDtypeStruct((H * V // 4, 128), jnp.float32),
        mesh=mesh,
        compiler_params=params,
        scratch_types=[
            pltpu.VMEM((R, VBLK + 1), jnp.float32),
            pltpu.VMEM((R, VBLK + 1), jnp.float32),
            pltpu.VMEM((VBLK * R // 128, 128), jnp.float32),
            pltpu.VMEM((VBLK * R // 128, 128), jnp.float32),
            pltpu.VMEM((H * R * VTAIL // 128, 128), jnp.float32),
            pltpu.SemaphoreType.DMA,
            pltpu.SemaphoreType.DMA,
        ],
    )
    tab2 = retile(tabT, tail)

    coords_flat = coords.reshape(B * H)
    offs = jnp.tile(jnp.arange(H, dtype=jnp.int32) * V, CHUNK_B)
    bias16 = jnp.broadcast_to(bias.astype(jnp.float32), (L,))
    run = pl.kernel(
        functools.partial(_cp_kernel, H, V, R, B),
        out_type=jax.ShapeDtypeStruct((B,), jnp.float32),
        mesh=mesh,
        compiler_params=params,
        scratch_types=[
            pltpu.VMEM((CHUNK_B * H,), jnp.int32),
            pltpu.VMEM((CHUNK_B * H,), jnp.int32),
            pltpu.VMEM((CHUNK_B * H // IDX_PER_STREAM, IDX_PER_STREAM),
                       jnp.int32),
            pltpu.VMEM((CHUNK_B * H,), jnp.int32),
            pltpu.VMEM((CHUNK_B * H, 4 * R), jnp.float32),
            pltpu.VMEM((R,), jnp.float32),
            pltpu.VMEM((L,), jnp.float32),
            pltpu.VMEM((per_tile,), jnp.float32),
            pltpu.SemaphoreType.DMA,
        ],
    )
    return run(coords_flat, offs, tab2, weights, bias16)


# speedup vs baseline: 1.7693x; 1.7693x over previous
import functools

import jax
import jax.numpy as jnp
from jax import lax
from jax.experimental import pallas as pl
from jax.experimental.pallas import tpu as pltpu
from jax.experimental.pallas import tpu_sc as plsc

L = 16
NUM_WORKERS = 32
CHUNK_B = 64
IDX_PER_STREAM = 128


def _cp_kernel(H, V, R, B, coords_hbm, offs_hbm, factors_hbm, weights_hbm,
               bias_hbm, out_hbm, cvm, offs_vm, fidx_vm, rows_vm, wvm, bvm,
               pbuf_vm, out_vm, sem):
    per_tile = B // NUM_WORKERS
    n_chunks = per_tile // CHUNK_B
    idx_per_chunk = CHUNK_B * H
    n_streams = idx_per_chunk // IDX_PER_STREAM
    n_groups = CHUNK_B // L

    wid = lax.axis_index("s") * 2 + lax.axis_index("c")

    pltpu.sync_copy(offs_hbm, offs_vm)
    pltpu.sync_copy(weights_hbm, wvm)
    pltpu.sync_copy(bias_hbm, bvm)

    iota = lax.iota(jnp.int32, L)
    iota17 = iota * (L + 1)
    bias_vec = bvm[...]
    w0 = wvm[pl.ds(0, L)]
    w1 = wvm[pl.ds(L, L)]

    def chunk_body(c):
        base = (wid * per_tile + c * CHUNK_B) * H
        pltpu.sync_copy(coords_hbm.at[pl.ds(base, idx_per_chunk)], cvm)
        for i in range(idx_per_chunk // L):
            j, col = (i * L) // IDX_PER_STREAM, (i * L) % IDX_PER_STREAM
            sl = pl.ds(i * L, L)
            fidx_vm[j, pl.ds(col, L)] = cvm[sl] + offs_vm[sl]
        copies = []
        for j in range(n_streams):
            copies.append(
                pltpu.async_copy(
                    factors_hbm.at[fidx_vm.at[j]],
                    rows_vm.at[pl.ds(j * IDX_PER_STREAM, IDX_PER_STREAM)],
                    sem,
                ))
        for cp in copies:
            cp.wait()

        def group_body(g):
            for b in range(L):
                row = (g * L + b) * H
                acc0 = rows_vm[row, pl.ds(0, L)]
                acc1 = rows_vm[row, pl.ds(L, L)]
                for m in range(1, H):
                    acc0 = acc0 * rows_vm[row + m, pl.ds(0, L)]
                    acc1 = acc1 * rows_vm[row + m, pl.ds(L, L)]
                pbuf_vm[pl.ds(b * (L + 1), L)] = acc0 * w0 + acc1 * w1
            out_vec = bias_vec
            for l in range(L):
                out_vec = out_vec + plsc.load_gather(pbuf_vm, [iota17 + l])
            out_vm[pl.ds(c * CHUNK_B + g * L, L)] = out_vec

        pl.loop(0, n_groups)(group_body)

    pl.loop(0, n_chunks)(chunk_body)
    pltpu.sync_copy(out_vm, out_hbm.at[pl.ds(wid * per_tile, per_tile)])


@jax.jit
def kernel(coords, factors, weights, bias):
    B, H = coords.shape
    _, V, R = factors.shape
    assert R == 32 and B % NUM_WORKERS == 0
    per_tile = B // NUM_WORKERS

    coords_flat = coords.reshape(B * H)
    factors_flat = factors.reshape(H * V, R)
    offs = jnp.tile(jnp.arange(H, dtype=jnp.int32) * V, CHUNK_B)
    bias16 = jnp.broadcast_to(bias.astype(jnp.float32), (L,))

    mesh = plsc.VectorSubcoreMesh(core_axis_name="c", subcore_axis_name="s",
                                  num_cores=2, num_subcores=16)
    run = pl.kernel(
        functools.partial(_cp_kernel, H, V, R, B),
        out_type=jax.ShapeDtypeStruct((B,), jnp.float32),
        mesh=mesh,
        compiler_params=pltpu.CompilerParams(needs_layout_passes=False,
                                             use_tc_tiling_on_sc=False),
        scratch_types=[
            pltpu.VMEM((CHUNK_B * H,), jnp.int32),
            pltpu.VMEM((CHUNK_B * H,), jnp.int32),
            pltpu.VMEM((CHUNK_B * H // IDX_PER_STREAM, IDX_PER_STREAM),
                       jnp.int32),
            pltpu.VMEM((CHUNK_B * H, R), jnp.float32),
            pltpu.VMEM((R,), jnp.float32),
            pltpu.VMEM((L,), jnp.float32),
            pltpu.VMEM((L * (L + 1),), jnp.float32),
            pltpu.VMEM((per_tile,), jnp.float32),
            pltpu.SemaphoreType.DMA,
        ],
    )
    return run(coords_flat, offs, factors_flat, weights, bias16)
